# Initial kernel scaffold; baseline (speedup 1.0000x reference)
import jax
import jax.numpy as jnp
from jax.experimental import pallas as pl


def kernel(inputs, W, Dk, Db, gamma, beta):
    raise NotImplementedError("write your pallas kernel here")


# speedup vs baseline: 2.7232x; 2.7232x over previous
import functools

import jax
import jax.numpy as jnp
from jax.experimental import pallas as pl

_LEN = 32
_HID = 8192
_K = 64
_OUT = 64


def _sort_stage(A, j, k):
    RB, n, L = A.shape
    G = n // (2 * j)
    A4 = A.reshape(RB, G, 2, j, L)
    a = A4[:, :, 0]
    b = A4[:, :, 1]
    mx = jnp.maximum(a, b)
    mn = jnp.minimum(a, b)
    shape = (RB, G, j, L)
    if k >= n:
        desc = True
    else:
        g = jax.lax.broadcasted_iota(jnp.int32, shape, 1)
        desc = ((g * (2 * j)) & k) == 0
    if L > 1:
        lane = jax.lax.broadcasted_iota(jnp.int32, shape, 3)
        want_desc = lane < (L // 2)
        dir_ = desc == want_desc if not isinstance(desc, bool) else want_desc
    else:
        dir_ = desc
    if isinstance(dir_, bool):
        new_a, new_b = mx, mn
    else:
        new_a = jnp.where(dir_, mx, mn)
        new_b = jnp.where(dir_, mn, mx)
    A4 = jnp.concatenate([new_a[:, :, None], new_b[:, :, None]], axis=2)
    return A4.reshape(RB, n, L)


def _bitonic_sort(A):
    n = A.shape[1]
    k = 2
    while k <= n:
        j = k // 2
        while j >= 1:
            A = _sort_stage(A, j, k)
            j //= 2
        k *= 2
    return A


def _bitonic_merge(A):
    n = A.shape[1]
    j = n // 2
    while j >= 1:
        A = _sort_stage(A, j, n)
        j //= 2
    return A


def _topk_merge_tree(A):
    L = A.shape[2]
    while L > 1:
        m = jnp.maximum(A[:, :, : L // 2], A[:, :, L // 2:])
        A = _bitonic_merge(m)
        L //= 2
    return A


def _body(x_ref, w_ref, dk_ref, db_ref, g_ref, b_ref, o_ref, *, rb):
    h = jnp.dot(x_ref[...], w_ref[...], preferred_element_type=jnp.float32)
    A = h.reshape(rb, _K, _HID // _K)
    A = _bitonic_sort(A)
    A = _topk_merge_tree(A)
    sg = A.reshape(rb, _K)
    d = jnp.dot(sg, dk_ref[...], preferred_element_type=jnp.float32)
    d = d + db_ref[...]
    d = 0.5 * d * (1.0 + jax.lax.erf(d * (2.0 ** -0.5)))
    mu = jnp.mean(d, axis=-1, keepdims=True)
    c = d - mu
    var = jnp.mean(c * c, axis=-1, keepdims=True)
    o_ref[...] = c * jax.lax.rsqrt(var + 1e-6) * g_ref[...] + b_ref[...]


def kernel(inputs, W, Dk, Db, gamma, beta):
    B, LEN = inputs.shape
    T = W.shape[0]
    rb = 128 if B % 128 == 0 else B
    grid = (T, B // rb)
    out = pl.pallas_call(
        functools.partial(_body, rb=rb),
        grid=grid,
        in_specs=[
            pl.BlockSpec((rb, LEN), lambda t, i: (i, 0)),
            pl.BlockSpec((None, LEN, _HID), lambda t, i: (t, 0, 0)),
            pl.BlockSpec((None, _K, _OUT), lambda t, i: (t, 0, 0)),
            pl.BlockSpec((None, 1, _OUT), lambda t, i: (t, 0, 0)),
            pl.BlockSpec((None, 1, _OUT), lambda t, i: (t, 0, 0)),
            pl.BlockSpec((None, 1, _OUT), lambda t, i: (t, 0, 0)),
        ],
        out_specs=pl.BlockSpec((None, rb, _OUT), lambda t, i: (t, i, 0)),
        out_shape=jax.ShapeDtypeStruct((T, B, _OUT), jnp.float32),
    )(inputs, W, Dk, Db[:, None, :], gamma[:, None, :], beta[:, None, :])
    return jnp.transpose(out, (1, 0, 2))


# speedup vs baseline: 23.8020x; 8.7405x over previous
import functools

import jax
import jax.numpy as jnp
from jax.experimental import pallas as pl

_LEN = 32
_HID = 8192
_K = 64
_OUT = 64
_NCOL = _HID // _K


def _ce(P, i, p):
    a, b = P[i], P[p]
    P[i] = jnp.maximum(a, b)
    P[p] = jnp.minimum(a, b)


def _bitonic_sort_positions(P):
    n = len(P)
    k = 2
    while k <= n:
        j = k // 2
        while j >= 1:
            for i in range(n):
                p = i ^ j
                if p > i:
                    if (i & k) == 0 or k == n:
                        _ce(P, i, p)
                    else:
                        _ce(P, p, i)
            j //= 2
        k *= 2
    return P


def _merge_positions(P):
    n = len(P)
    j = n // 2
    while j >= 1:
        for i in range(n):
            p = i ^ j
            if p > i:
                _ce(P, i, p)
        j //= 2
    return P


def _lane_mask(w, width):
    lm = jax.lax.broadcasted_iota(jnp.int32, (1, width), 1) < w
    return lm


def _body(x_ref, w_ref, dk_ref, db_ref, g_ref, b_ref, o_ref, *, rb):
    h = jnp.dot(x_ref[...], w_ref[...], preferred_element_type=jnp.float32)
    P = [h[:, _NCOL * i:_NCOL * (i + 1)] for i in range(_K)]
    lm = _lane_mask(_NCOL // 2, _NCOL)
    P = [jnp.where(lm, v, -v) for v in P]
    P = _bitonic_sort_positions(P)
    w = _NCOL
    while w > 1:
        half = w // 2
        P = [jnp.maximum(v[:, :half], -v[:, half:w]) for v in P]
        if half > 1:
            lm = _lane_mask(half // 2, half)
            P = [jnp.where(lm, v, -v) for v in P]
        P = _merge_positions(P)
        w = half
    sg = jnp.concatenate(P, axis=1)
    d = jnp.dot(sg, dk_ref[...], preferred_element_type=jnp.float32)
    d = d + db_ref[...]
    d = 0.5 * d * (1.0 + jax.lax.erf(d * (2.0 ** -0.5)))
    mu = jnp.mean(d, axis=-1, keepdims=True)
    c = d - mu
    var = jnp.mean(c * c, axis=-1, keepdims=True)
    o_ref[...] = c * jax.lax.rsqrt(var + 1e-6) * g_ref[...] + b_ref[...]


def kernel(inputs, W, Dk, Db, gamma, beta):
    B, LEN = inputs.shape
    T = W.shape[0]
    rb = 128 if B % 128 == 0 else B
    grid = (T, B // rb)
    out = pl.pallas_call(
        functools.partial(_body, rb=rb),
        grid=grid,
        in_specs=[
            pl.BlockSpec((rb, LEN), lambda t, i: (i, 0)),
            pl.BlockSpec((None, LEN, _HID), lambda t, i: (t, 0, 0)),
            pl.BlockSpec((None, _K, _OUT), lambda t, i: (t, 0, 0)),
            pl.BlockSpec((None, 1, _OUT), lambda t, i: (t, 0, 0)),
            pl.BlockSpec((None, 1, _OUT), lambda t, i: (t, 0, 0)),
            pl.BlockSpec((None, 1, _OUT), lambda t, i: (t, 0, 0)),
        ],
        out_specs=pl.BlockSpec((None, rb, _OUT), lambda t, i: (t, i, 0)),
        out_shape=jax.ShapeDtypeStruct((T, B, _OUT), jnp.float32),
    )(inputs, W, Dk, Db[:, None, :], gamma[:, None, :], beta[:, None, :])
    return jnp.transpose(out, (1, 0, 2))


# speedup vs baseline: 74.1223x; 3.1141x over previous
import functools

import jax
import jax.numpy as jnp
from jax.experimental import pallas as pl

_LEN = 32
_HID = 8192
_K = 64
_OUT = 64
_NCOL = _HID // _K


def _ce(P, i, p):
    a, b = P[i], P[p]
    P[i] = jnp.maximum(a, b)
    P[p] = jnp.minimum(a, b)


def _bitonic_sort_positions(P):
    n = len(P)
    k = 2
    while k <= n:
        j = k // 2
        while j >= 1:
            for i in range(n):
                p = i ^ j
                if p > i:
                    if (i & k) == 0 or k == n:
                        _ce(P, i, p)
                    else:
                        _ce(P, p, i)
            j //= 2
        k *= 2
    return P


def _merge_positions(P):
    n = len(P)
    j = n // 2
    while j >= 1:
        for i in range(n):
            p = i ^ j
            if p > i:
                _ce(P, i, p)
        j //= 2
    return P


def _sub_mask(w, width, lanes):
    return jax.lax.broadcasted_iota(jnp.int32, (width, lanes), 0) < w


def _body(x_ref, w_ref, dk_ref, db_ref, g_ref, b_ref, o_ref, *, rb):
    hT = jax.lax.dot_general(
        w_ref[...], x_ref[...],
        dimension_numbers=(((0,), (1,)), ((), ())),
        preferred_element_type=jnp.float32,
    )
    P = [hT[_NCOL * i:_NCOL * (i + 1), :] for i in range(_K)]
    lm = _sub_mask(_NCOL // 2, _NCOL, rb)
    P = [jnp.where(lm, v, -v) for v in P]
    P = _bitonic_sort_positions(P)
    w = _NCOL
    while w > 1:
        half = w // 2
        P = [jnp.maximum(v[:half, :], -v[half:w, :]) for v in P]
        if half > 1:
            lm = _sub_mask(half // 2, half, rb)
            P = [jnp.where(lm, v, -v) for v in P]
        P = _merge_positions(P)
        w = half
    sgT = jnp.concatenate(P, axis=0)
    dT = jax.lax.dot_general(
        dk_ref[...], sgT,
        dimension_numbers=(((0,), (0,)), ((), ())),
        preferred_element_type=jnp.float32,
    )
    dT = dT + db_ref[...]
    dT = 0.5 * dT * (1.0 + jax.lax.erf(dT * (2.0 ** -0.5)))
    mu = jnp.mean(dT, axis=0, keepdims=True)
    c = dT - mu
    var = jnp.mean(c * c, axis=0, keepdims=True)
    o_ref[...] = c * jax.lax.rsqrt(var + 1e-6) * g_ref[...] + b_ref[...]


def kernel(inputs, W, Dk, Db, gamma, beta):
    B, LEN = inputs.shape
    T = W.shape[0]
    rb = 128 if B % 128 == 0 else B
    grid = (T, B // rb)
    out = pl.pallas_call(
        functools.partial(_body, rb=rb),
        grid=grid,
        in_specs=[
            pl.BlockSpec((rb, LEN), lambda t, i: (i, 0)),
            pl.BlockSpec((None, LEN, _HID), lambda t, i: (t, 0, 0)),
            pl.BlockSpec((None, _K, _OUT), lambda t, i: (t, 0, 0)),
            pl.BlockSpec((None, _OUT, 1), lambda t, i: (t, 0, 0)),
            pl.BlockSpec((None, _OUT, 1), lambda t, i: (t, 0, 0)),
            pl.BlockSpec((None, _OUT, 1), lambda t, i: (t, 0, 0)),
        ],
        out_specs=pl.BlockSpec((None, _OUT, rb), lambda t, i: (t, 0, i)),
        out_shape=jax.ShapeDtypeStruct((T, _OUT, B), jnp.float32),
    )(inputs, W, Dk, Db[:, :, None], gamma[:, :, None], beta[:, :, None])
    return jnp.transpose(out, (2, 0, 1))
